# baseline (speedup 1.0000x reference)
import functools

import jax
import jax.numpy as jnp
import numpy as np
from jax import lax
from jax.experimental import pallas as pl
from jax.experimental.pallas import tpu as pltpu
from jax.experimental.pallas import tpu_sc as plsc

EPS = 1e-06


def _sc_gather_pack(table, idx, n_tokens, mm_dim, nw, chunk):
    n_chunks = n_tokens // (nw * chunk)
    mesh = plsc.VectorSubcoreMesh(core_axis_name="c", subcore_axis_name="s")
    nc = mesh.num_cores

    @functools.partial(
        pl.kernel,
        out_type=jax.ShapeDtypeStruct((n_tokens, mm_dim // 2), jnp.int32),
        mesh=mesh,
        scratch_types=[
            pltpu.VMEM((n_chunks, chunk), jnp.int32),
            pltpu.VMEM((chunk, mm_dim), jnp.float32),
            pltpu.VMEM((chunk, mm_dim), jnp.float32),
            pltpu.VMEM((chunk, mm_dim // 2), jnp.int32),
            pltpu.VMEM((chunk, mm_dim // 2), jnp.int32),
            pltpu.SemaphoreType.DMA,
            pltpu.SemaphoreType.DMA,
        ],
        compiler_params=pltpu.CompilerParams(needs_layout_passes=False),
    )
    def gather_kernel(table_hbm, idx_hbm, out_hbm,
                      idx_v, rows_a, rows_b, pk_a, pk_b, g_sem, o_sem):
        wid = lax.axis_index("s") * nc + lax.axis_index("c")
        base = wid * n_chunks * chunk
        fbufs = (rows_a, rows_b)
        bbufs = (pk_a, pk_b)

        def to_bf16_bits(v):
            u = plsc.bitcast(v, jnp.uint32)
            rnd = ((u >> 16) & 1) + 0x7FFF
            return (u + rnd) >> 16

        def pack_chunk(src, dst):
            def row_body(r, carry):
                def grp(j, carry2):
                    gbase = j * 32
                    a = src[r, pl.ds(gbase, 16)]
                    b = src[r, pl.ds(gbase + 16, 16)]
                    word = to_bf16_bits(a) | (to_bf16_bits(b) << 16)
                    dst[r, pl.ds(j * 16, 16)] = plsc.bitcast(word, jnp.int32)
                    return carry2
                return lax.fori_loop(0, mm_dim // 32, grp, carry)
            lax.fori_loop(0, chunk, row_body, 0)

        pltpu.sync_copy(idx_hbm.at[wid], idx_v)
        pltpu.async_copy(table_hbm.at[idx_v.at[0]], fbufs[0], g_sem)
        for c in range(n_chunks):
            fb = fbufs[c % 2]
            bb = bbufs[c % 2]
            pltpu.make_async_copy(table_hbm.at[idx_v.at[c]], fb, g_sem).wait()
            if c + 1 < n_chunks:
                pltpu.async_copy(
                    table_hbm.at[idx_v.at[c + 1]], fbufs[(c + 1) % 2], g_sem)
            if c >= 2:
                pltpu.make_async_copy(
                    bbufs[c % 2],
                    out_hbm.at[pl.ds(base + (c - 2) * chunk, chunk)],
                    o_sem).wait()
            pack_chunk(fb, bb)
            pltpu.async_copy(
                bb, out_hbm.at[pl.ds(base + c * chunk, chunk)], o_sem)
        for c in range(max(n_chunks - 2, 0), n_chunks):
            pltpu.make_async_copy(
                bbufs[c % 2],
                out_hbm.at[pl.ds(base + c * chunk, chunk)],
                o_sem).wait()

    return gather_kernel(table, idx)


def _tc_norm_proj_norm(emb, w_bf16, n_tokens, mm_dim, txt_dim, blk):
    def body(x_ref, w_ref, o_ref):
        x = x_ref[...]
        xf = x.astype(jnp.float32)
        m2 = jnp.mean(xf * xf, axis=-1, keepdims=True) + EPS
        z = lax.dot_general(
            x, w_ref[...], (((1,), (1,)), ((), ())),
            preferred_element_type=jnp.float32,
        )
        mz = jnp.mean(z * z, axis=-1, keepdims=True)
        o_ref[...] = z * lax.rsqrt(mz + EPS * m2)

    return pl.pallas_call(
        body,
        grid=(n_tokens // blk,),
        in_specs=[
            pl.BlockSpec((blk, mm_dim), lambda i: (i, 0)),
            pl.BlockSpec((txt_dim, mm_dim), lambda i: (0, 0)),
        ],
        out_specs=pl.BlockSpec((blk, txt_dim), lambda i: (i, 0)),
        out_shape=jax.ShapeDtypeStruct((n_tokens, txt_dim), jnp.float32),
    )(emb, w_bf16)


def _pack_perm(mm_dim):
    s = np.arange(mm_dim)
    return (s // 32) * 32 + (s % 2) * 16 + (s % 32) // 2


def kernel(input_ids, embedding_table, hard_norm_weight, proj_weight):
    b, s = input_ids.shape
    vocab, mm_dim = embedding_table.shape
    txt_dim = proj_weight.shape[0]
    n_tokens = b * s

    nw = 32
    chunk = 32
    perm = _pack_perm(mm_dim)
    w_eff = (proj_weight * hard_norm_weight)[:, perm].astype(jnp.bfloat16)
    ids = input_ids.reshape(nw, n_tokens // (nw * chunk), chunk).astype(jnp.int32)

    emb_i32 = _sc_gather_pack(embedding_table, ids, n_tokens, mm_dim, nw, chunk)
    emb = lax.bitcast_convert_type(emb_i32, jnp.bfloat16).reshape(n_tokens, mm_dim)
    out = _tc_norm_proj_norm(emb, w_eff, n_tokens, mm_dim, txt_dim, blk=512)
    return out.reshape(b, s, txt_dim)


# speedup vs baseline: 1.0675x; 1.0675x over previous
import functools

import jax
import jax.numpy as jnp
import numpy as np
from jax import lax
from jax.experimental import pallas as pl
from jax.experimental.pallas import tpu as pltpu
from jax.experimental.pallas import tpu_sc as plsc

EPS = 1e-06


def _sc_gather_pack(table, idx, n_tokens, mm_dim, nw, chunk):
    n_chunks = n_tokens // (nw * chunk)
    mesh = plsc.VectorSubcoreMesh(core_axis_name="c", subcore_axis_name="s")
    nc = mesh.num_cores

    @functools.partial(
        pl.kernel,
        out_type=jax.ShapeDtypeStruct((n_tokens, mm_dim // 2), jnp.int32),
        mesh=mesh,
        scratch_types=[
            pltpu.VMEM((n_chunks, chunk), jnp.int32),
            pltpu.VMEM((chunk, mm_dim), jnp.float32),
            pltpu.VMEM((chunk, mm_dim), jnp.float32),
            pltpu.VMEM((chunk, mm_dim // 2), jnp.int32),
            pltpu.VMEM((chunk, mm_dim // 2), jnp.int32),
            pltpu.SemaphoreType.DMA,
            pltpu.SemaphoreType.DMA,
        ],
        compiler_params=pltpu.CompilerParams(needs_layout_passes=False),
    )
    def gather_kernel(table_hbm, idx_hbm, out_hbm,
                      idx_v, rows_a, rows_b, pk_a, pk_b, g_sem, o_sem):
        wid = lax.axis_index("s") * nc + lax.axis_index("c")
        base = wid * n_chunks * chunk
        fbufs = (rows_a, rows_b)
        bbufs = (pk_a, pk_b)

        half = jnp.uint32(0x8000)
        himask = jnp.uint32(0xFFFF0000)

        def pack_chunk(src, dst):
            def row_body(r, carry):
                for j in range(mm_dim // 32):
                    gbase = j * 32
                    a = src[r, pl.ds(gbase, 16)]
                    b = src[r, pl.ds(gbase + 16, 16)]
                    ua = plsc.bitcast(a, jnp.uint32)
                    ub = plsc.bitcast(b, jnp.uint32)
                    lo = (ua + half) >> 16
                    hi = (ub + half) & himask
                    dst[r, pl.ds(j * 16, 16)] = plsc.bitcast(lo | hi, jnp.int32)
                return carry
            lax.fori_loop(0, chunk, row_body, 0)

        pltpu.sync_copy(idx_hbm.at[wid], idx_v)
        pltpu.async_copy(table_hbm.at[idx_v.at[0]], fbufs[0], g_sem)
        for c in range(n_chunks):
            fb = fbufs[c % 2]
            bb = bbufs[c % 2]
            pltpu.make_async_copy(table_hbm.at[idx_v.at[c]], fb, g_sem).wait()
            if c + 1 < n_chunks:
                pltpu.async_copy(
                    table_hbm.at[idx_v.at[c + 1]], fbufs[(c + 1) % 2], g_sem)
            if c >= 2:
                pltpu.make_async_copy(
                    bbufs[c % 2],
                    out_hbm.at[pl.ds(base + (c - 2) * chunk, chunk)],
                    o_sem).wait()
            pack_chunk(fb, bb)
            pltpu.async_copy(
                bb, out_hbm.at[pl.ds(base + c * chunk, chunk)], o_sem)
        for c in range(max(n_chunks - 2, 0), n_chunks):
            pltpu.make_async_copy(
                bbufs[c % 2],
                out_hbm.at[pl.ds(base + c * chunk, chunk)],
                o_sem).wait()

    return gather_kernel(table, idx)


def _tc_norm_proj_norm(emb, w_bf16, n_tokens, mm_dim, txt_dim, blk):
    def body(x_ref, w_ref, o_ref):
        x = x_ref[...]
        xf = x.astype(jnp.float32)
        m2 = jnp.mean(xf * xf, axis=-1, keepdims=True) + EPS
        z = lax.dot_general(
            x, w_ref[...], (((1,), (1,)), ((), ())),
            preferred_element_type=jnp.float32,
        )
        mz = jnp.mean(z * z, axis=-1, keepdims=True)
        o_ref[...] = z * lax.rsqrt(mz + EPS * m2)

    return pl.pallas_call(
        body,
        grid=(n_tokens // blk,),
        in_specs=[
            pl.BlockSpec((blk, mm_dim), lambda i: (i, 0)),
            pl.BlockSpec((txt_dim, mm_dim), lambda i: (0, 0)),
        ],
        out_specs=pl.BlockSpec((blk, txt_dim), lambda i: (i, 0)),
        out_shape=jax.ShapeDtypeStruct((n_tokens, txt_dim), jnp.float32),
    )(emb, w_bf16)


def _pack_perm(mm_dim):
    s = np.arange(mm_dim)
    return (s // 32) * 32 + (s % 2) * 16 + (s % 32) // 2


def kernel(input_ids, embedding_table, hard_norm_weight, proj_weight):
    b, s = input_ids.shape
    vocab, mm_dim = embedding_table.shape
    txt_dim = proj_weight.shape[0]
    n_tokens = b * s

    nw = 32
    chunk = 32
    perm = _pack_perm(mm_dim)
    w_eff = (proj_weight * hard_norm_weight)[:, perm].astype(jnp.bfloat16)
    ids = input_ids.reshape(nw, n_tokens // (nw * chunk), chunk).astype(jnp.int32)

    emb_i32 = _sc_gather_pack(embedding_table, ids, n_tokens, mm_dim, nw, chunk)
    emb = lax.bitcast_convert_type(emb_i32, jnp.bfloat16).reshape(n_tokens, mm_dim)
    out = _tc_norm_proj_norm(emb, w_eff, n_tokens, mm_dim, txt_dim, blk=512)
    return out.reshape(b, s, txt_dim)


# speedup vs baseline: 3.0270x; 2.8356x over previous
import functools

import jax
import jax.numpy as jnp
from jax import lax
from jax.experimental import pallas as pl
from jax.experimental.pallas import tpu as pltpu
from jax.experimental.pallas import tpu_sc as plsc

EPS = 1e-06


def _sc_gather(table, idx, n_tokens, mm_dim, nw, chunk):
    n_chunks = n_tokens // (nw * chunk)
    mesh = plsc.VectorSubcoreMesh(core_axis_name="c", subcore_axis_name="s")
    nc = mesh.num_cores

    @functools.partial(
        pl.kernel,
        out_type=jax.ShapeDtypeStruct((n_tokens, mm_dim), jnp.float32),
        mesh=mesh,
        scratch_types=[
            pltpu.VMEM((n_chunks, chunk), jnp.int32),
            pltpu.VMEM((chunk, mm_dim), jnp.float32),
            pltpu.VMEM((chunk, mm_dim), jnp.float32),
            pltpu.VMEM((chunk, mm_dim), jnp.float32),
            pltpu.SemaphoreType.DMA,
            pltpu.SemaphoreType.DMA,
        ],
    )
    def gather_kernel(table_hbm, idx_hbm, out_hbm,
                      idx_v, rows_a, rows_b, rows_c, g_sem, o_sem):
        wid = lax.axis_index("s") * nc + lax.axis_index("c")
        base = wid * n_chunks * chunk
        bufs = (rows_a, rows_b, rows_c)

        def gather(c):
            return pltpu.make_async_copy(
                table_hbm.at[idx_v.at[c]], bufs[c % 3], g_sem)

        def copyout(c):
            return pltpu.make_async_copy(
                bufs[c % 3], out_hbm.at[pl.ds(base + c * chunk, chunk)], o_sem)

        pltpu.sync_copy(idx_hbm.at[wid], idx_v)
        gather(0).start()
        gather(1).start()
        for c in range(n_chunks):
            gather(c).wait()
            copyout(c).start()
            g = c + 2
            if g < n_chunks:
                if g >= 3:
                    copyout(g - 3).wait()
                gather(g).start()
        for c in range(n_chunks - 3, n_chunks):
            copyout(c).wait()

    return gather_kernel(table, idx)


def _tc_norm_proj_norm(emb, w_bf16, n_tokens, mm_dim, txt_dim, blk):
    def body(x_ref, w_ref, o_ref):
        x = x_ref[...]
        m2 = jnp.mean(x * x, axis=-1, keepdims=True) + EPS
        z = lax.dot_general(
            x.astype(jnp.bfloat16), w_ref[...], (((1,), (1,)), ((), ())),
            preferred_element_type=jnp.float32,
        )
        mz = jnp.mean(z * z, axis=-1, keepdims=True)
        o_ref[...] = z * lax.rsqrt(mz + EPS * m2)

    return pl.pallas_call(
        body,
        grid=(n_tokens // blk,),
        in_specs=[
            pl.BlockSpec((blk, mm_dim), lambda i: (i, 0)),
            pl.BlockSpec((txt_dim, mm_dim), lambda i: (0, 0)),
        ],
        out_specs=pl.BlockSpec((blk, txt_dim), lambda i: (i, 0)),
        out_shape=jax.ShapeDtypeStruct((n_tokens, txt_dim), jnp.float32),
    )(emb, w_bf16)


def kernel(input_ids, embedding_table, hard_norm_weight, proj_weight):
    b, s = input_ids.shape
    vocab, mm_dim = embedding_table.shape
    txt_dim = proj_weight.shape[0]
    n_tokens = b * s

    nw = 32
    chunk = 32
    w_eff = (proj_weight * hard_norm_weight).astype(jnp.bfloat16)
    ids = input_ids.reshape(nw, n_tokens // (nw * chunk), chunk).astype(jnp.int32)

    emb = _sc_gather(embedding_table, ids, n_tokens, mm_dim, nw, chunk)
    out = _tc_norm_proj_norm(emb, w_eff, n_tokens, mm_dim, txt_dim, blk=512)
    return out.reshape(b, s, txt_dim)


# speedup vs baseline: 3.1537x; 1.0419x over previous
import functools

import jax
import jax.numpy as jnp
from jax import lax
from jax.experimental import pallas as pl
from jax.experimental.pallas import tpu as pltpu
from jax.experimental.pallas import tpu_sc as plsc

EPS = 1e-06


def _sc_gather(table, idx, n_tokens, mm_dim, nw, chunk):
    n_chunks = n_tokens // (nw * chunk)
    mesh = plsc.VectorSubcoreMesh(core_axis_name="c", subcore_axis_name="s")
    nc = mesh.num_cores

    @functools.partial(
        pl.kernel,
        out_type=jax.ShapeDtypeStruct((n_tokens, mm_dim), jnp.float32),
        mesh=mesh,
        scratch_types=[
            pltpu.VMEM((n_chunks, chunk), jnp.int32),
            pltpu.VMEM((chunk, mm_dim), jnp.float32),
            pltpu.VMEM((chunk, mm_dim), jnp.float32),
            pltpu.VMEM((chunk, mm_dim), jnp.float32),
            pltpu.SemaphoreType.DMA,
            pltpu.SemaphoreType.DMA,
        ],
    )
    def gather_kernel(table_hbm, idx_hbm, out_hbm,
                      idx_v, rows_a, rows_b, rows_c, g_sem, o_sem):
        wid = lax.axis_index("s") * nc + lax.axis_index("c")
        base = wid * n_chunks * chunk
        bufs = (rows_a, rows_b, rows_c)

        def gather(c):
            return pltpu.make_async_copy(
                table_hbm.at[idx_v.at[c]], bufs[c % 3], g_sem)

        def copyout(c):
            return pltpu.make_async_copy(
                bufs[c % 3], out_hbm.at[pl.ds(base + c * chunk, chunk)], o_sem)

        pltpu.sync_copy(idx_hbm.at[wid], idx_v)
        gather(0).start()
        gather(1).start()
        for c in range(n_chunks):
            gather(c).wait()
            copyout(c).start()
            g = c + 2
            if g < n_chunks:
                if g >= 3:
                    copyout(g - 3).wait()
                gather(g).start()
        for c in range(n_chunks - 3, n_chunks):
            copyout(c).wait()

    return gather_kernel(table, idx)


def _tc_norm_proj_norm(emb, w_bf16, n_tokens, mm_dim, txt_dim, blk):
    def body(x_ref, w_ref, o_ref):
        x = x_ref[...]
        m2 = jnp.mean(x * x, axis=-1, keepdims=True) + EPS
        z = lax.dot_general(
            x.astype(jnp.bfloat16), w_ref[...], (((1,), (1,)), ((), ())),
            preferred_element_type=jnp.float32,
        )
        mz = jnp.mean(z * z, axis=-1, keepdims=True)
        o_ref[...] = z * lax.rsqrt(mz + EPS * m2)

    return pl.pallas_call(
        body,
        grid=(n_tokens // blk,),
        in_specs=[
            pl.BlockSpec((blk, mm_dim), lambda i: (i, 0)),
            pl.BlockSpec((txt_dim, mm_dim), lambda i: (0, 0)),
        ],
        out_specs=pl.BlockSpec((blk, txt_dim), lambda i: (i, 0)),
        out_shape=jax.ShapeDtypeStruct((n_tokens, txt_dim), jnp.float32),
    )(emb, w_bf16)


def kernel(input_ids, embedding_table, hard_norm_weight, proj_weight):
    b, s = input_ids.shape
    vocab, mm_dim = embedding_table.shape
    txt_dim = proj_weight.shape[0]
    n_tokens = b * s

    nw = 32
    chunk = 32
    w_eff = (proj_weight * hard_norm_weight).astype(jnp.bfloat16)
    ids = input_ids.reshape(nw, n_tokens // (nw * chunk), chunk).astype(jnp.int32)

    emb = _sc_gather(embedding_table, ids, n_tokens, mm_dim, nw, chunk)
    out = _tc_norm_proj_norm(emb, w_eff, n_tokens, mm_dim, txt_dim, blk=1024)
    return out.reshape(b, s, txt_dim)
